# Initial kernel scaffold; baseline (speedup 1.0000x reference)
import jax
import jax.numpy as jnp
from jax.experimental import pallas as pl


def kernel(inputs, order2_table, order1_table):
    raise NotImplementedError("write your pallas kernel here")


# speedup vs baseline: 33.4932x; 33.4932x over previous
import functools

import jax
import jax.numpy as jnp
from jax import lax
from jax.experimental import pallas as pl
from jax.experimental.pallas import tpu as pltpu
from jax.experimental.pallas import tpu_sc as plsc

_VOCAB = 100000
_EMB = 128
_BATCH = 4096
_FIELDS = 100
_NC = 2
_NS = 16
_NW = _NC * _NS
_BPW = _BATCH // _NW
_FPAD = 104
_GRP = _BPW // 16
_VCHUNK = 4096
_VGRID = -(-_VOCAB // _VCHUNK)


def _c_table_body(t2_ref, t1_ref, c_ref):
    t2 = t2_ref[...]
    c_ref[...] = t1_ref[:, 0] - 0.5 * jnp.sum(t2 * t2, axis=1)


def _c_table(order2_table, order1_table):
    return pl.pallas_call(
        _c_table_body,
        grid=(_VGRID,),
        in_specs=[
            pl.BlockSpec((_VCHUNK, _EMB), lambda i: (i, 0)),
            pl.BlockSpec((_VCHUNK, 1), lambda i: (i, 0)),
        ],
        out_specs=pl.BlockSpec((_VCHUNK,), lambda i: (i,)),
        out_shape=jax.ShapeDtypeStruct((_VOCAB,), jnp.float32),
    )(order2_table, order1_table)


_mesh = plsc.VectorSubcoreMesh(
    core_axis_name="c", subcore_axis_name="s", num_cores=_NC, num_subcores=_NS
)


@functools.partial(
    pl.kernel,
    out_type=jax.ShapeDtypeStruct((_BATCH,), jnp.float32),
    mesh=_mesh,
    scratch_types=[
        pltpu.VMEM((_FPAD, _BPW), jnp.int32),
        pltpu.VMEM((_VOCAB,), jnp.float32),
        pltpu.VMEM((_BPW, _EMB), jnp.float32),
        pltpu.VMEM((_BPW,), jnp.float32),
        pltpu.SemaphoreType.DMA,
    ],
    compiler_params=pltpu.CompilerParams(needs_layout_passes=False),
)
def _fm_sc(idx_hbm, t2_hbm, c_hbm, out_hbm, idx_v, c_v, acc_v, out_v, gsem):
    cid = lax.axis_index("c")
    sid = lax.axis_index("s")
    wid = sid * _NC + cid
    base = wid * _BPW

    pltpu.sync_copy(idx_hbm.at[pl.ds(wid * _FPAD, _FPAD)], idx_v)
    pltpu.sync_copy(c_hbm, c_v)

    zeros16 = jnp.zeros((16,), jnp.float32)

    def _zero(b, carry):
        for j in range(_EMB // 16):
            acc_v[b, pl.ds(j * 16, 16)] = zeros16
        return carry

    lax.fori_loop(0, _BPW, _zero, 0)

    def _fire(f, carry):
        pltpu.async_copy(t2_hbm.at[idx_v.at[f]], acc_v, gsem, add=True)
        return carry

    lax.fori_loop(0, _FIELDS, _fire, 0)

    def _csum(f, cacc):
        out = []
        for g in range(_GRP):
            i16 = idx_v[f, pl.ds(g * 16, 16)]
            out.append(cacc[g] + plsc.load_gather(c_v, [i16]))
        return tuple(out)

    cacc = lax.fori_loop(0, _FIELDS, _csum, (zeros16,) * _GRP)

    def _drain(f, carry):
        pltpu.make_async_copy(t2_hbm.at[idx_v.at[0]], acc_v, gsem).wait()
        return carry

    lax.fori_loop(0, _FIELDS, _drain, 0)

    rows = [jnp.arange(16, dtype=jnp.int32) + g * 16 for g in range(_GRP)]

    def _ssq(d, ssq):
        col = jnp.full((16,), d, jnp.int32)
        out = []
        for g in range(_GRP):
            v = plsc.load_gather(acc_v, [rows[g], col])
            out.append(ssq[g] + v * v)
        return tuple(out)

    ssq = lax.fori_loop(0, _EMB, _ssq, (zeros16,) * _GRP)

    for g in range(_GRP):
        out_v[pl.ds(g * 16, 16)] = 0.5 * ssq[g] + cacc[g]

    pltpu.sync_copy(out_v, out_hbm.at[pl.ds(base, _BPW)])


def kernel(inputs, order2_table, order1_table):
    idx = inputs.astype(jnp.int32)
    idx_t = idx.T.reshape(_FIELDS, _NW, _BPW)
    slab = jnp.transpose(idx_t, (1, 0, 2))
    slab = jnp.pad(slab, ((0, 0), (0, _FPAD - _FIELDS), (0, 0)))
    slab = slab.reshape(_NW * _FPAD, _BPW)
    c = _c_table(order2_table, order1_table)
    out = _fm_sc(slab, order2_table, c)
    return out.reshape(_BATCH, 1)


# speedup vs baseline: 44.3150x; 1.3231x over previous
import functools

import jax
import jax.numpy as jnp
from jax import lax
from jax.experimental import pallas as pl
from jax.experimental.pallas import tpu as pltpu
from jax.experimental.pallas import tpu_sc as plsc

_VOCAB = 100000
_EMB = 128
_BATCH = 4096
_FIELDS = 100
_NC = 2
_NS = 16
_NW = _NC * _NS
_BPW = _BATCH // _NW
_FPAD = 104
_GRP = _BPW // 16
_VCHUNK = 4096
_VGRID = -(-_VOCAB // _VCHUNK)


def _c_table_body(t2_ref, t1r_ref, c_ref):
    t2 = t2_ref[...]
    ones = jnp.ones((1, _EMB), jnp.float32)
    norm2 = jax.lax.dot_general(
        ones, t2 * t2, (((1,), (1,)), ((), ())),
        preferred_element_type=jnp.float32,
    )
    c_ref[...] = t1r_ref[...] - 0.5 * norm2


def _c_table(order2_table, order1_row):
    return pl.pallas_call(
        _c_table_body,
        grid=(_VGRID,),
        in_specs=[
            pl.BlockSpec((_VCHUNK, _EMB), lambda i: (i, 0)),
            pl.BlockSpec((1, _VCHUNK), lambda i: (0, i)),
        ],
        out_specs=pl.BlockSpec((1, _VCHUNK), lambda i: (0, i)),
        out_shape=jax.ShapeDtypeStruct((1, _VOCAB), jnp.float32),
    )(order2_table, order1_row)


_mesh = plsc.VectorSubcoreMesh(
    core_axis_name="c", subcore_axis_name="s", num_cores=_NC, num_subcores=_NS
)


@functools.partial(
    pl.kernel,
    out_type=jax.ShapeDtypeStruct((_BATCH,), jnp.float32),
    mesh=_mesh,
    scratch_types=[
        pltpu.VMEM((_FPAD, _BPW), jnp.int32),
        pltpu.VMEM((_VOCAB,), jnp.float32),
        pltpu.VMEM((_BPW, _EMB), jnp.float32),
        pltpu.VMEM((_BPW,), jnp.float32),
        pltpu.SemaphoreType.DMA,
    ],
    compiler_params=pltpu.CompilerParams(needs_layout_passes=False),
)
def _fm_sc(idx_hbm, t2_hbm, c_hbm, out_hbm, idx_v, c_v, acc_v, out_v, gsem):
    cid = lax.axis_index("c")
    sid = lax.axis_index("s")
    wid = sid * _NC + cid
    base = wid * _BPW

    pltpu.sync_copy(idx_hbm.at[pl.ds(wid * _FPAD, _FPAD)], idx_v)
    pltpu.sync_copy(c_hbm, c_v)

    zeros16 = jnp.zeros((16,), jnp.float32)

    def _zero(b, carry):
        for j in range(_EMB // 16):
            acc_v[b, pl.ds(j * 16, 16)] = zeros16
        return carry

    lax.fori_loop(0, _BPW, _zero, 0)

    def _fire(f, carry):
        pltpu.async_copy(t2_hbm.at[idx_v.at[f]], acc_v, gsem, add=True)
        return carry

    lax.fori_loop(0, _FIELDS, _fire, 0)

    def _csum(f, cacc):
        out = []
        for g in range(_GRP):
            i16 = idx_v[f, pl.ds(g * 16, 16)]
            out.append(cacc[g] + plsc.load_gather(c_v, [i16]))
        return tuple(out)

    cacc = lax.fori_loop(0, _FIELDS, _csum, (zeros16,) * _GRP)

    def _drain(f, carry):
        pltpu.make_async_copy(t2_hbm.at[idx_v.at[0]], acc_v, gsem).wait()
        return carry

    lax.fori_loop(0, _FIELDS, _drain, 0)

    rows = [jnp.arange(16, dtype=jnp.int32) + g * 16 for g in range(_GRP)]

    def _ssq(d, ssq):
        col = jnp.full((16,), d, jnp.int32)
        out = []
        for g in range(_GRP):
            v = plsc.load_gather(acc_v, [rows[g], col])
            out.append(ssq[g] + v * v)
        return tuple(out)

    ssq = lax.fori_loop(0, _EMB, _ssq, (zeros16,) * _GRP)

    for g in range(_GRP):
        out_v[pl.ds(g * 16, 16)] = 0.5 * ssq[g] + cacc[g]

    pltpu.sync_copy(out_v, out_hbm.at[pl.ds(base, _BPW)])


def kernel(inputs, order2_table, order1_table):
    idx = inputs.astype(jnp.int32)
    idx_t = idx.T.reshape(_FIELDS, _NW, _BPW)
    slab = jnp.transpose(idx_t, (1, 0, 2))
    slab = jnp.pad(slab, ((0, 0), (0, _FPAD - _FIELDS), (0, 0)))
    slab = slab.reshape(_NW * _FPAD, _BPW)
    t1_row = order1_table.reshape(1, _VOCAB)
    c = _c_table(order2_table, t1_row).reshape(_VOCAB)
    out = _fm_sc(slab, order2_table, c)
    return out.reshape(_BATCH, 1)
